# baseline (speedup 1.0000x reference)
import functools

import jax
import jax.numpy as jnp
from jax import lax
from jax.experimental import pallas as pl
from jax.experimental.pallas import tpu as pltpu
from jax.experimental.pallas import tpu_sc as plsc

_BATCH = 4096
_IN_DIM = 8192
_OUT_DIM = 16384

_NW = 32
_ROWS_PER_W = _BATCH // _NW
_R = 8
_NBLK = _ROWS_PER_W // _R
_OC = 512
_NCH = _OUT_DIM // _OC
_GPC = _OC // 16
_OCHUNK = _R * _OC

_mesh = plsc.VectorSubcoreMesh(core_axis_name="c", subcore_axis_name="s")


@functools.partial(
    pl.kernel,
    out_type=jax.ShapeDtypeStruct((_BATCH * _OUT_DIM,), jnp.float32),
    mesh=_mesh,
    compiler_params=pltpu.CompilerParams(needs_layout_passes=False),
    scratch_types=[
        pltpu.VMEM((_OUT_DIM,), jnp.int32),
        pltpu.VMEM((_OUT_DIM,), jnp.int32),
        pltpu.VMEM((_OUT_DIM,), jnp.float32),
        pltpu.VMEM((_R * _IN_DIM,), jnp.float32),
        pltpu.VMEM((2 * _OCHUNK,), jnp.float32),
        pltpu.SemaphoreType.DMA,
    ],
)
def _logic_fwd(x_hbm, iaib_hbm, cacb_hbm, cab_hbm, out_hbm,
               iaib_v, cacb_v, cab_v, rows_v, ob_v, out_sem):
    wid = lax.axis_index("s") * 2 + lax.axis_index("c")
    pltpu.sync_copy(iaib_hbm, iaib_v)
    pltpu.sync_copy(cacb_hbm, cacb_v)
    pltpu.sync_copy(cab_hbm, cab_v)

    def out_drain():
        pltpu.make_async_copy(
            ob_v.at[pl.ds(0, _OCHUNK)],
            out_hbm.at[pl.ds(0, _OCHUNK)], out_sem).wait()

    def blk_body(blk, _):
        grp = wid * _NBLK + blk
        pltpu.sync_copy(
            x_hbm.at[pl.ds(grp * (_R * _IN_DIM), _R * _IN_DIM)], rows_v)

        def ch_body(ch, _):
            obuf = lax.rem(ch, 2)
            lin = blk * _NCH + ch

            @pl.when(lin >= 2)
            def _():
                out_drain()

            @plsc.parallel_loop(0, _GPC, 1, unroll=8)
            def g_body(g):
                gbase = ch * _OC + g * 16
                v = iaib_v[pl.ds(gbase, 16)]
                ia = lax.bitwise_and(v, 0xFFFF)
                ib = lax.shift_right_logical(v, 16)
                cc = cacb_v[pl.ds(gbase, 16)]
                ca = plsc.bitcast(lax.shift_left(cc, 16), jnp.float32)
                cb = plsc.bitcast(
                    lax.bitwise_and(cc, jnp.int32(-65536)), jnp.float32)
                cab = cab_v[pl.ds(gbase, 16)]
                obase = obuf * _OCHUNK + (g // 8) * 1024 + (g % 8) * 16
                for r in range(_R):
                    rowslc = rows_v.at[pl.ds(r * 128, _R * _IN_DIM - 896)]
                    a = plsc.load_gather(rowslc, [ia])
                    b = plsc.load_gather(rowslc, [ib])
                    ob_v[pl.ds(obase + r * 128, 16)] = (
                        a * (ca + cab * b) + cb * b)

            pltpu.async_copy(
                ob_v.at[pl.ds(obuf * _OCHUNK, _OCHUNK)],
                out_hbm.at[pl.ds(grp * (_R * _OUT_DIM) + ch * _OCHUNK,
                                 _OCHUNK)],
                out_sem)
            return 0

        lax.fori_loop(0, _NCH, ch_body, 0)
        return 0

    lax.fori_loop(0, _NBLK, blk_body, 0)
    out_drain()
    out_drain()


@jax.jit
def kernel(x, weights, idx_a, idx_b):
    w = jax.nn.softmax(weights, axis=-1)
    ca = w[:, 2] + w[:, 3]
    cb = w[:, 4] + w[:, 5]
    cab = w[:, 1] - w[:, 2] - w[:, 4]
    ta = ((idx_a >> 7) << 10) | (idx_a & 127)
    tb = ((idx_b >> 7) << 10) | (idx_b & 127)
    iaib = ta | (tb << 16)
    ca16 = jax.lax.bitcast_convert_type(
        ca.astype(jnp.bfloat16), jnp.uint16).astype(jnp.int32)
    cb16 = jax.lax.bitcast_convert_type(
        cb.astype(jnp.bfloat16), jnp.uint16).astype(jnp.int32)
    cacb = (cb16 << 16) | ca16
    x_lin = x.reshape(_BATCH // 8, 8, _IN_DIM // 128, 128)
    x_lin = x_lin.transpose(0, 2, 1, 3).reshape(-1)
    out = _logic_fwd(x_lin, iaib, cacb, cab)
    out = out.reshape(_BATCH // 8, _OUT_DIM // 128, 8, 128)
    out = out.transpose(0, 2, 1, 3).reshape(_BATCH, _OUT_DIM)
    return out


# speedup vs baseline: 1.1318x; 1.1318x over previous
import functools

import jax
import jax.numpy as jnp
from jax import lax
from jax.experimental import pallas as pl
from jax.experimental.pallas import tpu as pltpu
from jax.experimental.pallas import tpu_sc as plsc

_BATCH = 4096
_IN_DIM = 8192
_OUT_DIM = 16384

_NW = 32
_ROWS_PER_W = _BATCH // _NW
_R = 8
_NBLK = _ROWS_PER_W // _R
_OC = 1024
_NCH = _OUT_DIM // _OC
_GP2 = _OC // 32
_OCHUNK = _R * _OC

_mesh = plsc.VectorSubcoreMesh(core_axis_name="c", subcore_axis_name="s")


@functools.partial(
    pl.kernel,
    out_type=jax.ShapeDtypeStruct((_BATCH * _OUT_DIM,), jnp.float32),
    mesh=_mesh,
    compiler_params=pltpu.CompilerParams(needs_layout_passes=False),
    scratch_types=[
        pltpu.VMEM((_OUT_DIM,), jnp.int32),
        pltpu.VMEM((_OUT_DIM,), jnp.int32),
        pltpu.VMEM((_OUT_DIM // 2,), jnp.int32),
        pltpu.VMEM((_R * _IN_DIM,), jnp.float32),
        pltpu.VMEM((2 * _OCHUNK,), jnp.float32),
        pltpu.SemaphoreType.DMA,
    ],
)
def _logic_fwd(x_hbm, iaib_hbm, cacb_hbm, cabp_hbm, out_hbm,
               iaib_v, cacb_v, cabp_v, rows_v, ob_v, out_sem):
    wid = lax.axis_index("s") * 2 + lax.axis_index("c")
    pltpu.sync_copy(iaib_hbm, iaib_v)
    pltpu.sync_copy(cacb_hbm, cacb_v)
    pltpu.sync_copy(cabp_hbm, cabp_v)

    def out_drain():
        pltpu.make_async_copy(
            ob_v.at[pl.ds(0, _OCHUNK)],
            out_hbm.at[pl.ds(0, _OCHUNK)], out_sem).wait()

    def blk_body(blk, _):
        grp = wid * _NBLK + blk
        pltpu.sync_copy(
            x_hbm.at[pl.ds(grp * (_R * _IN_DIM), _R * _IN_DIM)], rows_v)

        def ch_body(ch, _):
            obuf = lax.rem(ch, 2)
            lin = blk * _NCH + ch

            @pl.when(lin >= 2)
            def _():
                out_drain()

            @plsc.parallel_loop(0, _GP2, 1, unroll=2)
            def gp_body(t):
                gbase = ch * _OC + t * 32
                cc2 = cabp_v[pl.ds(ch * (_OC // 2) + t * 16, 16)]
                cab2 = (plsc.bitcast(lax.shift_left(cc2, 16), jnp.float32),
                        plsc.bitcast(lax.bitwise_and(cc2, jnp.int32(-65536)),
                                     jnp.float32))
                for h in range(2):
                    g = t * 2 + h
                    v = iaib_v[pl.ds(gbase + h * 16, 16)]
                    ia = lax.bitwise_and(v, 0xFFFF)
                    ib = lax.shift_right_logical(v, 16)
                    cc = cacb_v[pl.ds(gbase + h * 16, 16)]
                    ca = plsc.bitcast(lax.shift_left(cc, 16), jnp.float32)
                    cb = plsc.bitcast(
                        lax.bitwise_and(cc, jnp.int32(-65536)), jnp.float32)
                    cab = cab2[h]
                    obase = obuf * _OCHUNK + (g // 8) * 1024 + (g % 8) * 16
                    for r in range(_R):
                        rowslc = rows_v.at[
                            pl.ds(r * 128, _R * _IN_DIM - 896)]
                        a = plsc.load_gather(rowslc, [ia])
                        b = plsc.load_gather(rowslc, [ib])
                        ob_v[pl.ds(obase + r * 128, 16)] = (
                            a * (ca + cab * b) + cb * b)

            pltpu.async_copy(
                ob_v.at[pl.ds(obuf * _OCHUNK, _OCHUNK)],
                out_hbm.at[pl.ds(grp * (_R * _OUT_DIM) + ch * _OCHUNK,
                                 _OCHUNK)],
                out_sem)
            return 0

        lax.fori_loop(0, _NCH, ch_body, 0)
        return 0

    lax.fori_loop(0, _NBLK, blk_body, 0)
    out_drain()
    out_drain()


def _bf16_bits(v):
    return jax.lax.bitcast_convert_type(
        v.astype(jnp.bfloat16), jnp.uint16).astype(jnp.int32)


@jax.jit
def kernel(x, weights, idx_a, idx_b):
    w = jax.nn.softmax(weights, axis=-1)
    ca = w[:, 2] + w[:, 3]
    cb = w[:, 4] + w[:, 5]
    cab = w[:, 1] - w[:, 2] - w[:, 4]
    ta = ((idx_a >> 7) << 10) | (idx_a & 127)
    tb = ((idx_b >> 7) << 10) | (idx_b & 127)
    iaib = ta | (tb << 16)
    cacb = (_bf16_bits(cb) << 16) | _bf16_bits(ca)
    cabr = _bf16_bits(cab).reshape(-1, 2, 16)
    cabp = ((cabr[:, 1, :] << 16) | cabr[:, 0, :]).reshape(-1)
    x_lin = x.reshape(_BATCH // 8, 8, _IN_DIM // 128, 128)
    x_lin = x_lin.transpose(0, 2, 1, 3).reshape(-1)
    out = _logic_fwd(x_lin, iaib, cacb, cabp)
    out = out.reshape(_BATCH // 8, _OUT_DIM // 128, 8, 128)
    out = out.transpose(0, 2, 1, 3).reshape(_BATCH, _OUT_DIM)
    return out
